# baseline (speedup 1.0000x reference)
import functools

import jax
import jax.numpy as jnp
from jax import lax
from jax.experimental import pallas as pl
from jax.experimental.pallas import tpu as pltpu
from jax.experimental.pallas import tpu_sc as plsc

B = 1024
L = 256
NP = 10
D = 200
NT = 4

NC = 2
NS = 16
NW = NC * NS
EPW = B // NW


def _sc_body(task_id_hbm, patch_hbm, prompt_hbm, out_hbm,
             tid_v, buf0, buf1, in_sem0, in_sem1, out_sem0, out_sem1):
    bufs = (buf0, buf1)
    in_sems = (in_sem0, in_sem1)
    out_sems = (out_sem0, out_sem1)

    wid = lax.axis_index("s") * NC + lax.axis_index("c")
    base = wid * EPW

    pltpu.sync_copy(task_id_hbm.at[pl.ds(base, EPW)], tid_v)

    def tid_of(e):
        g, k = divmod(e, 16)
        vec = tid_v[pl.ds(g * 16, 16)]
        return vec[k]

    def start_in(e):
        b = e % 2
        i = base + e
        tid = tid_of(e)
        cp = pltpu.async_copy(prompt_hbm.at[pl.ds(tid * (NP * D), NP * D)],
                              bufs[b].at[pl.ds(0, NP * D)],
                              in_sems[b])
        cq = pltpu.async_copy(patch_hbm.at[pl.ds(i * (L * D), L * D)],
                              bufs[b].at[pl.ds(NP * D, L * D)],
                              in_sems[b])
        return cp, cq

    pending_in = start_in(0)
    pending_out = [None, None]
    for e in range(EPW):
        b = e % 2
        if e + 1 < EPW:
            if pending_out[1 - b] is not None:
                pending_out[1 - b].wait()
            next_in = start_in(e + 1)
        else:
            next_in = None
        pending_in[0].wait()
        pending_in[1].wait()
        row = (NP + L) * D
        pending_out[b] = pltpu.async_copy(
            bufs[b], out_hbm.at[pl.ds((base + e) * row, row)], out_sems[b])
        pending_in = next_in
    pending_out[0].wait()
    pending_out[1].wait()


@jax.jit
def _sc_concat(task_id, patch_embeddings, prompt_tokens):
    mesh = plsc.VectorSubcoreMesh(core_axis_name="c", subcore_axis_name="s")
    fn = functools.partial(
        pl.kernel,
        mesh=mesh,
        out_type=jax.ShapeDtypeStruct((B * (NP + L) * D,), jnp.float32),
        scratch_types=[
            pltpu.VMEM((EPW,), jnp.int32),
            pltpu.VMEM(((NP + L) * D,), jnp.float32),
            pltpu.VMEM(((NP + L) * D,), jnp.float32),
            pltpu.SemaphoreType.DMA,
            pltpu.SemaphoreType.DMA,
            pltpu.SemaphoreType.DMA,
            pltpu.SemaphoreType.DMA,
        ],
    )(_sc_body)
    out = fn(task_id,
             patch_embeddings.reshape(B * L * D),
             prompt_tokens.reshape(NT * NP * D))
    return out.reshape(B, NP + L, D)


def kernel(task_id, patch_embeddings, prompt_tokens):
    return _sc_concat(task_id.astype(jnp.int32), patch_embeddings,
                      prompt_tokens)
